# baseline (speedup 1.0000x reference)
import functools

import jax
import jax.numpy as jnp
from jax import lax
from jax.experimental import pallas as pl
from jax.experimental.pallas import tpu as pltpu
from jax.experimental.pallas import tpu_sc as plsc

B, C, H, W = 2, 96, 384, 384
N = H * W
CC = 8
G = C // CC
NT = 16
NPT = N // NT
S = 128
NCH = NPT // S
ROWS = 4 * S
NIDX = ROWS // 128
ZB = NPT // ROWS
WBR = ROWS * CC // 128
OUTR = N * CC // 128
PAD = 8
F32 = jnp.float32
I32 = jnp.int32


def _sc_body(frame, flow, out, acc, stage, data,
             idx0, idx1, idx2, idx3, ibuf, flow_v, wbuf):
    b = lax.axis_index("c")
    s = lax.axis_index("s")
    row0 = s * NPT
    iota = lax.broadcasted_iota(I32, (16,), 0)

    def fill_ibuf(base):
        def f_body(t, carry):
            w = t * 16 + iota
            plsc.store_scatter(ibuf, [w], base + w)
            return carry

        lax.fori_loop(0, ROWS // 16, f_body, 0)

    def pass_body(g, carry):
        zvec = jnp.zeros((16,), F32)

        def z_body(t, carry2):
            w = t * 16 + iota
            plsc.store_scatter(data, [w // CC, w % CC], zvec)
            return carry2

        lax.fori_loop(0, ROWS * CC // 16, z_body, 0)

        def zc_body(j, carry2):
            fill_ibuf(row0 + j * ROWS)
            pltpu.sync_copy(data, acc.at[ibuf])
            return carry2

        lax.fori_loop(0, ZB, zc_body, 0)
        plsc.subcore_barrier()

        def chunk_body(ci, carry2):
            blk = s * NCH + ci
            pltpu.sync_copy(frame.at[b, pl.ds(g * CC, CC), blk, :], stage)
            pltpu.sync_copy(flow.at[b, :, blk, :], flow_v)

            idxbufs = (idx0, idx1, idx2, idx3)
            for k in range(S // 16):
                fx = flow_v[0, pl.ds(k * 16, 16)]
                fy = flow_v[1, pl.ds(k * 16, 16)]
                p = row0 + ci * S + k * 16 + iota
                tx = (p % W).astype(F32) + fx
                ty = (p // W).astype(F32) + fy
                x0 = tx.astype(I32)
                x0 = x0 - jnp.where(x0.astype(F32) > tx, 1, 0)
                y0 = ty.astype(I32)
                y0 = y0 - jnp.where(y0.astype(F32) > ty, 1, 0)
                wx1 = tx - x0.astype(F32)
                wx0 = 1.0 - wx1
                wy1 = ty - y0.astype(F32)
                wy0 = 1.0 - wy1
                x1 = x0 + 1
                y1 = y0 + 1
                zero = jnp.zeros((16,), F32)
                mx0 = (x0 >= 0) & (x0 < W)
                mx1 = (x1 >= 0) & (x1 < W)
                my0 = (y0 >= 0) & (y0 < H)
                my1 = (y1 >= 0) & (y1 < H)
                w00 = jnp.where(mx0 & my0, wx0 * wy0, zero)
                w10 = jnp.where(mx1 & my0, wx1 * wy0, zero)
                w01 = jnp.where(mx0 & my1, wx0 * wy1, zero)
                w11 = jnp.where(mx1 & my1, wx1 * wy1, zero)
                swap = x0 == -1
                wa0 = jnp.where(swap, w10, w00)
                wa1 = jnp.where(swap, zero, w10)
                wb0 = jnp.where(swap, w11, w01)
                wb1 = jnp.where(swap, zero, w11)
                bx = jnp.clip(x0, 0, W - 1)
                ia = jnp.clip(y0, 0, H - 1) * W + bx
                ib = jnp.clip(y1, 0, H - 1) * W + bx
                ibk = idxbufs[k // 2]
                cbase = 64 * (k % 2) + 4 * iota
                plsc.store_scatter(ibk, [cbase], ia)
                plsc.store_scatter(ibk, [cbase + 1], ia + 1)
                plsc.store_scatter(ibk, [cbase + 2], ib)
                plsc.store_scatter(ibk, [cbase + 3], ib + 1)
                rbase = 64 * k + 4 * iota
                for c in range(CC):
                    v = stage[c, pl.ds(k * 16, 16)]
                    ccol = jnp.full((16,), c, I32)
                    plsc.store_scatter(data, [rbase, ccol], wa0 * v)
                    plsc.store_scatter(data, [rbase + 1, ccol], wa1 * v)
                    plsc.store_scatter(data, [rbase + 2, ccol], wb0 * v)
                    plsc.store_scatter(data, [rbase + 3, ccol], wb1 * v)

            for j in range(NIDX):
                pltpu.sync_copy(data.at[pl.ds(j * 128, 128), :],
                                acc.at[idxbufs[j]], add=True)
            return carry2

        lax.fori_loop(0, NCH, chunk_body, 0)
        plsc.subcore_barrier()

        def wb_body(j, carry2):
            fill_ibuf(row0 + j * ROWS)
            pltpu.sync_copy(acc.at[ibuf], data)

            def rp_body(t, carry3):
                w = t * 16 + iota
                v = plsc.load_gather(data, [w // CC, w % CC])
                wbuf[t // 8, pl.ds((t % 8) * 16, 16)] = v
                return carry3

            lax.fori_loop(0, ROWS * CC // 16, rp_body, 0)
            pltpu.sync_copy(
                wbuf,
                out.at[b, g, pl.ds(s * (NPT * CC // 128) + j * WBR, WBR), :])
            return carry2

        lax.fori_loop(0, ZB, wb_body, 0)
        return carry

    lax.fori_loop(0, G, pass_body, 0)


_splat_sc = functools.partial(
    pl.kernel,
    out_type=jax.ShapeDtypeStruct((B, G, OUTR, 128), F32),
    mesh=plsc.VectorSubcoreMesh(core_axis_name="c", subcore_axis_name="s",
                                num_cores=2, num_subcores=16),
    scratch_types=[
        pltpu.VMEM_SHARED((N + PAD, CC), F32),
        pltpu.VMEM((CC, S), F32),
        pltpu.VMEM((ROWS, CC), F32),
        pltpu.VMEM((128,), I32),
        pltpu.VMEM((128,), I32),
        pltpu.VMEM((128,), I32),
        pltpu.VMEM((128,), I32),
        pltpu.VMEM((ROWS,), I32),
        pltpu.VMEM((2, S), F32),
        pltpu.VMEM((WBR, 128), F32),
    ],
    compiler_params=pltpu.CompilerParams(use_tc_tiling_on_sc=False,
                                         needs_layout_passes=False),
)(_sc_body)


def kernel(frame, flow):
    frame_r = frame.reshape(B, C, N // S, S)
    flow_r = flow.reshape(B, 2, N // S, S)
    out = _splat_sc(frame_r, flow_r).reshape(B, G, N, CC)
    return out.transpose(0, 1, 3, 2).reshape(B, C, H, W)


# speedup vs baseline: 1.1945x; 1.1945x over previous
import functools

import jax
import jax.numpy as jnp
from jax import lax
from jax.experimental import pallas as pl
from jax.experimental.pallas import tpu as pltpu
from jax.experimental.pallas import tpu_sc as plsc

B, C, H, W = 2, 96, 384, 384
N = H * W
CC = 8
G = C // CC
NT = 16
NPT = N // NT
S = 512
NCH = NPT // S
ROWS = 4 * S
BLK = 1024
ZB = NPT // BLK
WBR = BLK * CC // 128
OUTR = N * CC // 128
PAD = 8
F32 = jnp.float32
I32 = jnp.int32


def _sc_body(frame, flow, out, acc, stage, data, idxc, ibuf, flow_v, wbuf):
    b = lax.axis_index("c")
    s = lax.axis_index("s")
    row0 = s * NPT
    iota = lax.broadcasted_iota(I32, (16,), 0)

    def fill_ibuf(base):
        def f_body(t, carry):
            w = t * 16 + iota
            plsc.store_scatter(ibuf, [w], base + w)
            return carry

        lax.fori_loop(0, BLK // 16, f_body, 0)

    def pass_body(g, carry):
        zvec = jnp.zeros((16,), F32)

        def z_body(t, carry2):
            w = t * 16 + iota
            plsc.store_scatter(data, [w // CC, w % CC], zvec)
            return carry2

        lax.fori_loop(0, ROWS * CC // 16, z_body, 0)

        def zc_body(j, carry2):
            fill_ibuf(row0 + j * BLK)
            pltpu.sync_copy(data.at[pl.ds(0, BLK), :], acc.at[ibuf])
            return carry2

        lax.fori_loop(0, ZB, zc_body, 0)
        plsc.subcore_barrier()

        def chunk_body(ci, carry2):
            blk = s * NCH + ci
            pltpu.sync_copy(frame.at[b, pl.ds(g * CC, CC), blk, :], stage)
            pltpu.sync_copy(flow.at[b, :, blk, :], flow_v)

            def k_body(k, carry3):
                fx = flow_v[0, pl.ds(k * 16, 16)]
                fy = flow_v[1, pl.ds(k * 16, 16)]
                p = row0 + ci * S + k * 16 + iota
                tx = (p % W).astype(F32) + fx
                ty = (p // W).astype(F32) + fy
                x0 = tx.astype(I32)
                x0 = x0 - jnp.where(x0.astype(F32) > tx, 1, 0)
                y0 = ty.astype(I32)
                y0 = y0 - jnp.where(y0.astype(F32) > ty, 1, 0)
                wx1 = tx - x0.astype(F32)
                wx0 = 1.0 - wx1
                wy1 = ty - y0.astype(F32)
                wy0 = 1.0 - wy1
                x1 = x0 + 1
                y1 = y0 + 1
                zero = jnp.zeros((16,), F32)
                mx0 = (x0 >= 0) & (x0 < W)
                mx1 = (x1 >= 0) & (x1 < W)
                my0 = (y0 >= 0) & (y0 < H)
                my1 = (y1 >= 0) & (y1 < H)
                w00 = jnp.where(mx0 & my0, wx0 * wy0, zero)
                w10 = jnp.where(mx1 & my0, wx1 * wy0, zero)
                w01 = jnp.where(mx0 & my1, wx0 * wy1, zero)
                w11 = jnp.where(mx1 & my1, wx1 * wy1, zero)
                swap = x0 == -1
                wa0 = jnp.where(swap, w10, w00)
                wa1 = jnp.where(swap, zero, w10)
                wb0 = jnp.where(swap, w11, w01)
                wb1 = jnp.where(swap, zero, w11)
                bx = jnp.clip(x0, 0, W - 1)
                ia = jnp.clip(y0, 0, H - 1) * W + bx
                ib = jnp.clip(y1, 0, H - 1) * W + bx
                cbase = 64 * k + 4 * iota
                plsc.store_scatter(idxc, [cbase], ia)
                plsc.store_scatter(idxc, [cbase + 1], ia + 1)
                plsc.store_scatter(idxc, [cbase + 2], ib)
                plsc.store_scatter(idxc, [cbase + 3], ib + 1)
                rbase = 64 * k + 4 * iota
                for c in range(CC):
                    v = stage[c, pl.ds(k * 16, 16)]
                    ccol = jnp.full((16,), c, I32)
                    plsc.store_scatter(data, [rbase, ccol], wa0 * v)
                    plsc.store_scatter(data, [rbase + 1, ccol], wa1 * v)
                    plsc.store_scatter(data, [rbase + 2, ccol], wb0 * v)
                    plsc.store_scatter(data, [rbase + 3, ccol], wb1 * v)
                return carry3

            lax.fori_loop(0, S // 16, k_body, 0)
            pltpu.sync_copy(data, acc.at[idxc], add=True)
            return carry2

        lax.fori_loop(0, NCH, chunk_body, 0)
        plsc.subcore_barrier()

        def wb_body(j, carry2):
            fill_ibuf(row0 + j * BLK)
            pltpu.sync_copy(acc.at[ibuf], data.at[pl.ds(0, BLK), :])

            def rp_body(t, carry3):
                w = t * 16 + iota
                v = plsc.load_gather(data, [w // CC, w % CC])
                wbuf[t // 8, pl.ds((t % 8) * 16, 16)] = v
                return carry3

            lax.fori_loop(0, BLK * CC // 16, rp_body, 0)
            pltpu.sync_copy(
                wbuf,
                out.at[b, g, pl.ds(s * (NPT * CC // 128) + j * WBR, WBR), :])
            return carry2

        lax.fori_loop(0, ZB, wb_body, 0)
        return carry

    lax.fori_loop(0, G, pass_body, 0)


_splat_sc = functools.partial(
    pl.kernel,
    out_type=jax.ShapeDtypeStruct((B, G, OUTR, 128), F32),
    mesh=plsc.VectorSubcoreMesh(core_axis_name="c", subcore_axis_name="s",
                                num_cores=2, num_subcores=16),
    scratch_types=[
        pltpu.VMEM_SHARED((N + PAD, CC), F32),
        pltpu.VMEM((CC, S), F32),
        pltpu.VMEM((ROWS, CC), F32),
        pltpu.VMEM((ROWS,), I32),
        pltpu.VMEM((BLK,), I32),
        pltpu.VMEM((2, S), F32),
        pltpu.VMEM((WBR, 128), F32),
    ],
    compiler_params=pltpu.CompilerParams(use_tc_tiling_on_sc=False,
                                         needs_layout_passes=False),
)(_sc_body)


def kernel(frame, flow):
    frame_r = frame.reshape(B, C, N // S, S)
    flow_r = flow.reshape(B, 2, N // S, S)
    out = _splat_sc(frame_r, flow_r).reshape(B, G, N, CC)
    return out.transpose(0, 1, 3, 2).reshape(B, C, H, W)


# speedup vs baseline: 1.2320x; 1.0313x over previous
import functools

import jax
import jax.numpy as jnp
from jax import lax
from jax.experimental import pallas as pl
from jax.experimental.pallas import tpu as pltpu
from jax.experimental.pallas import tpu_sc as plsc

B, C, H, W = 2, 96, 384, 384
N = H * W
CC = 8
G = C // CC
NT = 16
NPT = N // NT
S = 512
NCH = NPT // S
ROWS = 4 * S
BLK = 1024
ZB = NPT // BLK
WBR = BLK * CC // 128
OUTR = N * CC // 128
PAD = 8
F32 = jnp.float32
I32 = jnp.int32


def _sc_body(frame, flow, out, acc, stage, data, data1, idxc, idxc1,
             ibuf, flow_v, wbuf, sem0, sem1):
    b = lax.axis_index("c")
    s = lax.axis_index("s")
    row0 = s * NPT
    iota = lax.broadcasted_iota(I32, (16,), 0)

    def fill_ibuf(base):
        def f_body(t, carry):
            w = t * 16 + iota
            plsc.store_scatter(ibuf, [w], base + w)
            return carry

        lax.fori_loop(0, BLK // 16, f_body, 0)

    def pass_body(g, carry):
        zvec = jnp.zeros((16,), F32)

        def z_body(t, carry2):
            w = t * 16 + iota
            plsc.store_scatter(data, [w // CC, w % CC], zvec)
            return carry2

        lax.fori_loop(0, ROWS * CC // 16, z_body, 0)

        def zc_body(j, carry2):
            fill_ibuf(row0 + j * BLK)
            pltpu.sync_copy(data.at[pl.ds(0, BLK), :], acc.at[ibuf])
            return carry2

        lax.fori_loop(0, ZB, zc_body, 0)
        plsc.subcore_barrier()

        def do_chunk(ci, dbuf, ibufc, sem, first):
            blk = s * NCH + ci
            pltpu.sync_copy(frame.at[b, pl.ds(g * CC, CC), blk, :], stage)
            pltpu.sync_copy(flow.at[b, :, blk, :], flow_v)

            @pl.when(jnp.logical_not(first))
            def _():
                pltpu.make_async_copy(dbuf, acc.at[ibufc], sem).wait()

            def k_body(k, carry3):
                fx = flow_v[0, pl.ds(k * 16, 16)]
                fy = flow_v[1, pl.ds(k * 16, 16)]
                p = row0 + ci * S + k * 16 + iota
                tx = (p % W).astype(F32) + fx
                ty = (p // W).astype(F32) + fy
                x0 = tx.astype(I32)
                x0 = x0 - jnp.where(x0.astype(F32) > tx, 1, 0)
                y0 = ty.astype(I32)
                y0 = y0 - jnp.where(y0.astype(F32) > ty, 1, 0)
                wx1 = tx - x0.astype(F32)
                wx0 = 1.0 - wx1
                wy1 = ty - y0.astype(F32)
                wy0 = 1.0 - wy1
                x1 = x0 + 1
                y1 = y0 + 1
                zero = jnp.zeros((16,), F32)
                mx0 = (x0 >= 0) & (x0 < W)
                mx1 = (x1 >= 0) & (x1 < W)
                my0 = (y0 >= 0) & (y0 < H)
                my1 = (y1 >= 0) & (y1 < H)
                w00 = jnp.where(mx0 & my0, wx0 * wy0, zero)
                w10 = jnp.where(mx1 & my0, wx1 * wy0, zero)
                w01 = jnp.where(mx0 & my1, wx0 * wy1, zero)
                w11 = jnp.where(mx1 & my1, wx1 * wy1, zero)
                swap = x0 == -1
                wa0 = jnp.where(swap, w10, w00)
                wa1 = jnp.where(swap, zero, w10)
                wb0 = jnp.where(swap, w11, w01)
                wb1 = jnp.where(swap, zero, w11)
                bx = jnp.clip(x0, 0, W - 1)
                ia = jnp.clip(y0, 0, H - 1) * W + bx
                ib = jnp.clip(y1, 0, H - 1) * W + bx
                cbase = 64 * k + 4 * iota
                plsc.store_scatter(ibufc, [cbase], ia)
                plsc.store_scatter(ibufc, [cbase + 1], ia + 1)
                plsc.store_scatter(ibufc, [cbase + 2], ib)
                plsc.store_scatter(ibufc, [cbase + 3], ib + 1)
                rbase = 64 * k + 4 * iota
                for c in range(CC):
                    v = stage[c, pl.ds(k * 16, 16)]
                    ccol = jnp.full((16,), c, I32)
                    plsc.store_scatter(dbuf, [rbase, ccol], wa0 * v)
                    plsc.store_scatter(dbuf, [rbase + 1, ccol], wa1 * v)
                    plsc.store_scatter(dbuf, [rbase + 2, ccol], wb0 * v)
                    plsc.store_scatter(dbuf, [rbase + 3, ccol], wb1 * v)
                return carry3

            lax.fori_loop(0, S // 16, k_body, 0)
            pltpu.async_copy(dbuf, acc.at[ibufc], sem, add=True)

        def chunk_body(jj, carry2):
            do_chunk(2 * jj, data, idxc, sem0, jj == 0)
            do_chunk(2 * jj + 1, data1, idxc1, sem1, jj == 0)
            return carry2

        lax.fori_loop(0, NCH // 2, chunk_body, 0)
        pltpu.make_async_copy(data, acc.at[idxc], sem0).wait()
        pltpu.make_async_copy(data1, acc.at[idxc1], sem1).wait()
        plsc.subcore_barrier()

        def wb_body(j, carry2):
            fill_ibuf(row0 + j * BLK)
            pltpu.sync_copy(acc.at[ibuf], data.at[pl.ds(0, BLK), :])

            def rp_body(t, carry3):
                w = t * 16 + iota
                v = plsc.load_gather(data, [w // CC, w % CC])
                wbuf[t // 8, pl.ds((t % 8) * 16, 16)] = v
                return carry3

            lax.fori_loop(0, BLK * CC // 16, rp_body, 0)
            pltpu.sync_copy(
                wbuf,
                out.at[b, g, pl.ds(s * (NPT * CC // 128) + j * WBR, WBR), :])
            return carry2

        lax.fori_loop(0, ZB, wb_body, 0)
        return carry

    lax.fori_loop(0, G, pass_body, 0)


_splat_sc = functools.partial(
    pl.kernel,
    out_type=jax.ShapeDtypeStruct((B, G, OUTR, 128), F32),
    mesh=plsc.VectorSubcoreMesh(core_axis_name="c", subcore_axis_name="s",
                                num_cores=2, num_subcores=16),
    scratch_types=[
        pltpu.VMEM_SHARED((N + PAD, CC), F32),
        pltpu.VMEM((CC, S), F32),
        pltpu.VMEM((ROWS, CC), F32),
        pltpu.VMEM((ROWS, CC), F32),
        pltpu.VMEM((ROWS,), I32),
        pltpu.VMEM((ROWS,), I32),
        pltpu.VMEM((BLK,), I32),
        pltpu.VMEM((2, S), F32),
        pltpu.VMEM((WBR, 128), F32),
        pltpu.SemaphoreType.DMA,
        pltpu.SemaphoreType.DMA,
    ],
    compiler_params=pltpu.CompilerParams(use_tc_tiling_on_sc=False,
                                         needs_layout_passes=False),
)(_sc_body)


def kernel(frame, flow):
    frame_r = frame.reshape(B, C, N // S, S)
    flow_r = flow.reshape(B, 2, N // S, S)
    out = _splat_sc(frame_r, flow_r).reshape(B, G, N, CC)
    return out.transpose(0, 1, 3, 2).reshape(B, C, H, W)
